# Initial kernel scaffold; baseline (speedup 1.0000x reference)
import jax
import jax.numpy as jnp
from jax.experimental import pallas as pl


def kernel(output, anchors, targets):
    raise NotImplementedError("write your pallas kernel here")


# speedup vs baseline: 1.6983x; 1.6983x over previous
import functools

import jax
import jax.numpy as jnp
from jax.experimental import pallas as pl
from jax.experimental.pallas import tpu as pltpu

_THRESHOLD = 0.5


def _body(t_ref, an_ref, x_ref, out_ref, *, A, H, W, T_total, T_per_b, n_elems):
    i = pl.program_id(0)
    n = pl.num_programs(0)
    b_id = i // A
    a_id = i % A

    pred = x_ref[0, :, :, 4]

    tx = t_ref[1:2, :]
    ty = t_ref[2:3, :]
    tw = t_ref[3:4, :] * float(W)
    th = t_ref[4:5, :] * float(H)
    area_t = tw * th

    best_iou = jnp.full_like(tx, -jnp.inf)
    best_a = jnp.zeros_like(tx, dtype=jnp.int32)
    for k in range(A):
        aw = an_ref[k:k + 1, 0:1]
        ah = an_ref[k:k + 1, 1:2]
        inter = jnp.minimum(aw, tw) * jnp.minimum(ah, th)
        iou = inter / (aw * ah + area_t - inter)
        if k == 0:
            best_iou = iou
        else:
            upd = iou > best_iou
            best_a = jnp.where(upd, k, best_a)
            best_iou = jnp.where(upd, iou, best_iou)

    t_i = (tx * float(W)).astype(jnp.int32)
    t_j = (ty * float(H)).astype(jnp.int32)
    t_b = jax.lax.broadcasted_iota(jnp.int32, tx.shape, 1) // T_per_b

    hit = ((best_iou > _THRESHOLD)
           & (t_b == b_id)
           & (best_a == a_id)).astype(jnp.float32)

    row_iota = jax.lax.broadcasted_iota(jnp.int32, (H, T_total), 0)
    col_iota = jax.lax.broadcasted_iota(jnp.int32, (W, T_total), 0)
    oj = (row_iota == t_j).astype(jnp.float32) * hit
    oi = (col_iota == t_i).astype(jnp.float32)
    cnt = jax.lax.dot_general(oj, oi, (((1,), (1,)), ((), ())),
                              preferred_element_type=jnp.float32)
    gt = cnt > 0.0

    log_p = jnp.maximum(jnp.log(pred), -100.0)
    log_1p = jnp.maximum(jnp.log(1.0 - pred), -100.0)
    s = jnp.sum(jnp.where(gt, -log_p, -log_1p))

    acc = jnp.where(i == 0, 0.0, out_ref[0, 0]) + s
    out_ref[0, 0] = jnp.where(i == n - 1, acc / float(n_elems), acc)


def kernel(output, anchors, targets):
    B, A, H, W, C = output.shape
    T = targets.shape[1]
    n_elems = B * A * H * W
    x = output.reshape(B * A, H, W, C)
    tt = targets.reshape(B * T, 5).T

    out = pl.pallas_call(
        functools.partial(_body, A=A, H=H, W=W, T_total=B * T, T_per_b=T,
                          n_elems=n_elems),
        grid=(B * A,),
        in_specs=[
            pl.BlockSpec((5, B * T), lambda i: (0, 0)),
            pl.BlockSpec((A, 2), lambda i: (0, 0)),
            pl.BlockSpec((1, H, W, C), lambda i: (i, 0, 0, 0)),
        ],
        out_specs=pl.BlockSpec(memory_space=pltpu.SMEM),
        out_shape=jax.ShapeDtypeStruct((1, 1), jnp.float32),
    )(tt, anchors, x)
    return out[0, 0]


# speedup vs baseline: 3.1952x; 1.8814x over previous
import functools

import jax
import jax.numpy as jnp
from jax.experimental import pallas as pl
from jax.experimental.pallas import tpu as pltpu

_THRESHOLD = 0.5


def _body(t_ref, an_ref, x_ref, out_ref, *, BS, A, H, W, T_total, T_per_b,
          n_elems):
    i = pl.program_id(0)
    n = pl.num_programs(0)

    pred = x_ref[:, :, :, 4].reshape(BS * H, W)

    tx = t_ref[1:2, :]
    ty = t_ref[2:3, :]
    tw = t_ref[3:4, :] * float(W)
    th = t_ref[4:5, :] * float(H)
    area_t = tw * th

    best_iou = None
    best_a = jnp.zeros_like(tx, dtype=jnp.int32)
    for k in range(A):
        aw = an_ref[k:k + 1, 0:1]
        ah = an_ref[k:k + 1, 1:2]
        inter = jnp.minimum(aw, tw) * jnp.minimum(ah, th)
        iou = inter / (aw * ah + area_t - inter)
        if k == 0:
            best_iou = iou
        else:
            upd = iou > best_iou
            best_a = jnp.where(upd, k, best_a)
            best_iou = jnp.where(upd, iou, best_iou)

    t_i = (tx * float(W)).astype(jnp.int32)
    t_j = (ty * float(H)).astype(jnp.int32)
    t_b = jax.lax.broadcasted_iota(jnp.int32, tx.shape, 1) // T_per_b
    matched = best_iou > _THRESHOLD

    row_iota = jax.lax.broadcasted_iota(jnp.int32, (H, T_total), 0)
    col_iota = jax.lax.broadcasted_iota(jnp.int32, (W, T_total), 0)
    oj_base = (row_iota == t_j)
    oi = (col_iota == t_i).astype(jnp.float32)

    oj_rows = []
    for s in range(BS):
        plane = i * BS + s
        b_id = plane // A
        a_id = plane % A
        hit = (matched & (t_b == b_id) & (best_a == a_id)).astype(jnp.float32)
        oj_rows.append(oj_base.astype(jnp.float32) * hit)
    oj = jnp.concatenate(oj_rows, axis=0)
    cnt = jax.lax.dot_general(oj, oi, (((1,), (1,)), ((), ())),
                              preferred_element_type=jnp.float32)
    gt = cnt > 0.0

    log_p = jnp.maximum(jnp.log(pred), -100.0)
    log_1p = jnp.maximum(jnp.log(1.0 - pred), -100.0)
    s_sum = jnp.sum(jnp.where(gt, -log_p, -log_1p))

    acc = jnp.where(i == 0, 0.0, out_ref[0, 0]) + s_sum
    out_ref[0, 0] = jnp.where(i == n - 1, acc / float(n_elems), acc)


def kernel(output, anchors, targets):
    B, A, H, W, C = output.shape
    T = targets.shape[1]
    n_elems = B * A * H * W
    BS = 4
    x = output.reshape(B * A, H, W, C)
    tt = targets.reshape(B * T, 5).T

    out = pl.pallas_call(
        functools.partial(_body, BS=BS, A=A, H=H, W=W, T_total=B * T,
                          T_per_b=T, n_elems=n_elems),
        grid=(B * A // BS,),
        in_specs=[
            pl.BlockSpec((5, B * T), lambda i: (0, 0)),
            pl.BlockSpec((A, 2), lambda i: (0, 0)),
            pl.BlockSpec((BS, H, W, C), lambda i: (i, 0, 0, 0)),
        ],
        out_specs=pl.BlockSpec(memory_space=pltpu.SMEM),
        out_shape=jax.ShapeDtypeStruct((1, 1), jnp.float32),
    )(tt, anchors, x)
    return out[0, 0]


# speedup vs baseline: 3.4814x; 1.0896x over previous
import functools

import jax
import jax.numpy as jnp
from jax.experimental import pallas as pl
from jax.experimental.pallas import tpu as pltpu

_THRESHOLD = 0.5


def _body(t_ref, an_ref, x_ref, out_ref, *, BS, A, H, W, T_total, T_per_b,
          n_elems):
    i = pl.program_id(0)
    n = pl.num_programs(0)

    pred = x_ref[:, :, :, 4].reshape(BS * H, W)

    tx = t_ref[1:2, :]
    ty = t_ref[2:3, :]
    tw = t_ref[3:4, :] * float(W)
    th = t_ref[4:5, :] * float(H)
    area_t = tw * th

    best_iou = None
    best_a = jnp.zeros_like(tx, dtype=jnp.int32)
    for k in range(A):
        aw = an_ref[k:k + 1, 0:1]
        ah = an_ref[k:k + 1, 1:2]
        inter = jnp.minimum(aw, tw) * jnp.minimum(ah, th)
        iou = inter / (aw * ah + area_t - inter)
        if k == 0:
            best_iou = iou
        else:
            upd = iou > best_iou
            best_a = jnp.where(upd, k, best_a)
            best_iou = jnp.where(upd, iou, best_iou)

    t_i = (tx * float(W)).astype(jnp.int32)
    t_j = (ty * float(H)).astype(jnp.int32)
    t_b = jax.lax.broadcasted_iota(jnp.int32, tx.shape, 1) // T_per_b
    matched = best_iou > _THRESHOLD

    row_iota = jax.lax.broadcasted_iota(jnp.int32, (H, T_total), 0)
    col_iota = jax.lax.broadcasted_iota(jnp.int32, (W, T_total), 0)
    oj_base = (row_iota == t_j)
    oi = (col_iota == t_i).astype(jnp.float32)

    oj_rows = []
    for s in range(BS):
        plane = i * BS + s
        b_id = plane // A
        a_id = plane % A
        hit = (matched & (t_b == b_id) & (best_a == a_id)).astype(jnp.float32)
        oj_rows.append(oj_base.astype(jnp.float32) * hit)
    oj = jnp.concatenate(oj_rows, axis=0)
    cnt = jax.lax.dot_general(oj, oi, (((1,), (1,)), ((), ())),
                              preferred_element_type=jnp.float32)
    gt = cnt > 0.0

    log_p = jnp.maximum(jnp.log(pred), -100.0)
    log_1p = jnp.maximum(jnp.log(1.0 - pred), -100.0)
    s_sum = jnp.sum(jnp.where(gt, -log_p, -log_1p))

    acc = jnp.where(i == 0, 0.0, out_ref[0, 0]) + s_sum
    out_ref[0, 0] = jnp.where(i == n - 1, acc / float(n_elems), acc)


def kernel(output, anchors, targets):
    B, A, H, W, C = output.shape
    T = targets.shape[1]
    n_elems = B * A * H * W
    BS = 8
    x = output.reshape(B * A, H, W, C)
    tt = targets.reshape(B * T, 5).T

    out = pl.pallas_call(
        functools.partial(_body, BS=BS, A=A, H=H, W=W, T_total=B * T,
                          T_per_b=T, n_elems=n_elems),
        grid=(B * A // BS,),
        in_specs=[
            pl.BlockSpec((5, B * T), lambda i: (0, 0)),
            pl.BlockSpec((A, 2), lambda i: (0, 0)),
            pl.BlockSpec((BS, H, W, C), lambda i: (i, 0, 0, 0)),
        ],
        out_specs=pl.BlockSpec(memory_space=pltpu.SMEM),
        out_shape=jax.ShapeDtypeStruct((1, 1), jnp.float32),
    )(tt, anchors, x)
    return out[0, 0]
